# Initial kernel scaffold; baseline (speedup 1.0000x reference)
import jax
import jax.numpy as jnp
from jax.experimental import pallas as pl


def kernel(in_idx, tok_emb, pos_emb, W_out):
    raise NotImplementedError("write your pallas kernel here")


# speedup vs baseline: 1.1110x; 1.1110x over previous
import functools

import jax
import jax.numpy as jnp
from jax import lax
from jax.experimental import pallas as pl
from jax.experimental.pallas import tpu as pltpu
from jax.experimental.pallas import tpu_sc as plsc


def _sc_gather(table, idx_flat, n_tokens, emb):
    info = plsc.get_sparse_core_info()
    nw = info.num_cores * info.num_subcores
    assert n_tokens % (8 * nw) == 0
    b_per_w = n_tokens // nw
    nc = info.num_cores

    mesh = plsc.VectorSubcoreMesh(core_axis_name="c", subcore_axis_name="s")

    @functools.partial(
        pl.kernel,
        mesh=mesh,
        out_type=jax.ShapeDtypeStruct((n_tokens, emb), jnp.float32),
        scratch_types=[
            pltpu.VMEM((b_per_w,), jnp.int32),
            pltpu.VMEM((b_per_w, emb), jnp.float32),
            pltpu.SemaphoreType.DMA,
        ],
    )
    def gather_kernel(table_hbm, idx_hbm, out_hbm, idx_v, rows_v, sem):
        wid = lax.axis_index("s") * nc + lax.axis_index("c")
        base = wid * b_per_w
        pltpu.sync_copy(idx_hbm.at[pl.ds(base, b_per_w)], idx_v)
        pltpu.async_copy(table_hbm.at[idx_v], rows_v, sem).wait()
        pltpu.sync_copy(rows_v, out_hbm.at[pl.ds(base, b_per_w)])

    return gather_kernel(table, idx_flat)


def _tc_matmul(x_tok, pos_emb, w_out, batch, seq, vt):
    n = batch * seq
    emb = x_tok.shape[1]
    vocab = w_out.shape[0]
    n_vt = pl.cdiv(vocab, vt)

    def mm_kernel(x_ref, pos_ref, w_ref, out_ref, xs_ref):
        @pl.when(pl.program_id(0) == 0)
        def _():
            for b in range(batch):
                xs_ref[b * seq:(b + 1) * seq, :] = (
                    x_ref[b * seq:(b + 1) * seq, :] + pos_ref[...]
                ).astype(jnp.bfloat16)

        w_bf = w_ref[...].astype(jnp.bfloat16)
        out_ref[...] = lax.dot_general(
            xs_ref[...], w_bf,
            dimension_numbers=(((1,), (1,)), ((), ())),
            preferred_element_type=jnp.float32,
        )

    return pl.pallas_call(
        mm_kernel,
        grid=(n_vt,),
        in_specs=[
            pl.BlockSpec((n, emb), lambda v: (0, 0)),
            pl.BlockSpec((seq, emb), lambda v: (0, 0)),
            pl.BlockSpec((vt, emb), lambda v: (v, 0)),
        ],
        out_specs=pl.BlockSpec((n, vt), lambda v: (0, v)),
        out_shape=jax.ShapeDtypeStruct((n, vocab), jnp.float32),
        scratch_shapes=[pltpu.VMEM((n, emb), jnp.bfloat16)],
        compiler_params=pltpu.CompilerParams(
            dimension_semantics=("arbitrary",),
        ),
    )(x_tok, pos_emb, w_out)


def kernel(in_idx, tok_emb, pos_emb, W_out):
    batch, seq = in_idx.shape
    vocab, emb = W_out.shape
    idx_flat = in_idx.reshape(-1)
    x_tok = _sc_gather(tok_emb, idx_flat, batch * seq, emb)
    logits = _tc_matmul(x_tok, pos_emb[:seq], W_out, batch, seq, vt=512)
    return logits.reshape(batch, seq, vocab)
